# baseline (speedup 1.0000x reference)
import jax
import jax.numpy as jnp
from jax import lax
from jax.experimental import pallas as pl
from jax.experimental.pallas import tpu as pltpu
from jax.experimental.pallas import tpu_sc as plsc

_B = 16384
_F = 26
_E = 16
_V = 1000000

_NC = 2
_NS = 16
_NW = _NC * _NS
_BW = _B // _NW

_TC = 1024
_NCHUNK = _V // _TC
_TAIL = _V - _NCHUNK * _TC
_CPT = _NCHUNK // _NS


def _sc_body(idT_hbm, valT_hbm, wt_hbm, out_hbm, wrm_hbm,
             tbufA, tbufB, tobuf, idbuf, valbuf, rowsA, rowsB, obuf,
             semA, semB):
    c = lax.axis_index("c")
    s = lax.axis_index("s")
    wid = c * _NS + s
    b0 = wid * _BW
    lanes = lax.iota(jnp.int32, 16)

    def col0(j):
        return (j * _NS + s) * _TC

    def transpose_chunk(tbuf, n, cbase):
        def r_body(r, carry):
            vec = plsc.load_gather(tbuf, [lanes, jnp.full((16,), r, jnp.int32)])
            tobuf[r, :] = vec
            return carry
        lax.fori_loop(0, n, r_body, 0)
        pltpu.sync_copy(tobuf.at[pl.ds(0, n), :], wrm_hbm.at[c, pl.ds(cbase, n), :])

    pltpu.async_copy(wt_hbm.at[:, pl.ds(col0(0), _TC)], tbufA, semA)

    def pair_body(j, carry):
        f0 = 2 * j
        pltpu.async_copy(wt_hbm.at[:, pl.ds(col0(f0 + 1), _TC)], tbufB, semB)
        pltpu.make_async_copy(wt_hbm.at[:, pl.ds(0, _TC)], tbufA, semA).wait()
        transpose_chunk(tbufA, _TC, col0(f0))

        @pl.when(f0 + 2 < _CPT)
        def _():
            pltpu.async_copy(wt_hbm.at[:, pl.ds(col0(f0 + 2), _TC)], tbufA, semA)

        pltpu.make_async_copy(wt_hbm.at[:, pl.ds(0, _TC)], tbufB, semB).wait()
        transpose_chunk(tbufB, _TC, col0(f0 + 1))
        return carry

    lax.fori_loop(0, (_CPT - 1) // 2, pair_body, 0)
    pltpu.make_async_copy(wt_hbm.at[:, pl.ds(0, _TC)], tbufA, semA).wait()
    transpose_chunk(tbufA, _TC, col0(_CPT - 1))

    @pl.when(s == 0)
    def _():
        pltpu.sync_copy(wt_hbm.at[:, pl.ds(_NCHUNK * _TC, _TAIL)],
                        tbufA.at[:, pl.ds(0, _TAIL)])
        transpose_chunk(tbufA, _TAIL, _NCHUNK * _TC)

    plsc.subcore_barrier()

    pltpu.sync_copy(idT_hbm.at[:, pl.ds(b0, _BW)], idbuf)
    pltpu.sync_copy(valT_hbm.at[:, pl.ds(b0, _BW)], valbuf)

    def gather_f(f, rows, sem):
        pltpu.async_copy(wrm_hbm.at[c].at[idbuf.at[f]], rows, sem)

    def compute_f(f, rows):
        def g_body(g, carry):
            bidx = g * 16 + lanes
            v16 = valbuf[f, pl.ds(g * 16, 16)]
            for e in range(_E):
                col = jnp.full((16,), e, jnp.int32)
                obuf[e, pl.ds(g * 16, 16)] = plsc.load_gather(rows, [bidx, col]) * v16
            return carry
        lax.fori_loop(0, _BW // 16, g_body, 0)
        pltpu.sync_copy(obuf, out_hbm.at[f, :, pl.ds(b0, _BW)])

    gather_f(0, rowsA, semA)

    def f_pair(i, carry):
        fA = 2 * i
        gather_f(fA + 1, rowsB, semB)
        pltpu.make_async_copy(wrm_hbm.at[c].at[idbuf.at[0]], rowsA, semA).wait()
        compute_f(fA, rowsA)

        @pl.when(fA + 2 < _F)
        def _():
            gather_f(fA + 2, rowsA, semA)

        pltpu.make_async_copy(wrm_hbm.at[c].at[idbuf.at[0]], rowsB, semB).wait()
        compute_f(fA + 1, rowsB)
        return carry

    lax.fori_loop(0, _F // 2, f_pair, 0)


def kernel(id, value, W):
    idT = id.T
    valT = value.T
    wT = W.T
    mesh = plsc.VectorSubcoreMesh(core_axis_name="c", subcore_axis_name="s")
    outT, _ = pl.kernel(
        _sc_body,
        mesh=mesh,
        compiler_params=pltpu.CompilerParams(
            use_tc_tiling_on_sc=False, needs_layout_passes=False),
        out_type=(
            jax.ShapeDtypeStruct((_F, _E, _B), jnp.float32),
            jax.ShapeDtypeStruct((_NC, _V, _E), jnp.float32),
        ),
        scratch_types=[
            pltpu.VMEM((_E, _TC), jnp.float32),
            pltpu.VMEM((_E, _TC), jnp.float32),
            pltpu.VMEM((_TC, _E), jnp.float32),
            pltpu.VMEM((_F, _BW), jnp.int32),
            pltpu.VMEM((_F, _BW), jnp.float32),
            pltpu.VMEM((_BW, _E), jnp.float32),
            pltpu.VMEM((_BW, _E), jnp.float32),
            pltpu.VMEM((_E, _BW), jnp.float32),
            pltpu.SemaphoreType.DMA,
            pltpu.SemaphoreType.DMA,
        ],
    )(idT, valT, wT)
    return outT.transpose(2, 0, 1)


# speedup vs baseline: 2.2834x; 2.2834x over previous
import jax
import jax.numpy as jnp
from jax import lax
from jax.experimental import pallas as pl
from jax.experimental.pallas import tpu as pltpu
from jax.experimental.pallas import tpu_sc as plsc

_B = 16384
_F = 26
_E = 16
_V = 1000000
_VP = 1000064

_NC = 2
_NS = 16
_NW = _NC * _NS
_BW = _B // _NW

_TC = 1024
_NCHUNK = _V // _TC
_TAIL = _V - _NCHUNK * _TC
_CPT = _NCHUNK // _NS


def _transpose_body(wt_hbm, tail_hbm, wrm_hbm, tbufA, tbufB, tobuf, semA, semB):
    c = lax.axis_index("c")
    s = lax.axis_index("s")
    lanes = lax.iota(jnp.int32, 16)

    def col0(j):
        return (j * _NS + s) * _TC

    def transpose_chunk(tbuf, n, cbase):
        def i_body(i, carry):
            for jj in range(8):
                q = i * 8 + jj
                vec = plsc.load_gather(
                    tbuf, [lanes, jnp.full((16,), q, jnp.int32)])
                tobuf[i, pl.ds(jj * 16, 16)] = vec
            return carry
        lax.fori_loop(0, n * _E // 128, i_body, 0)
        r0 = pl.multiple_of(cbase * _E // 128, 8)
        pltpu.sync_copy(
            tobuf.at[pl.ds(0, n * _E // 128), :],
            wrm_hbm.at[c, pl.ds(r0, n * _E // 128), :])

    pltpu.async_copy(wt_hbm.at[:, pl.ds(col0(0), _TC)], tbufA, semA)

    def pair_body(j, carry):
        f0 = 2 * j
        pltpu.async_copy(wt_hbm.at[:, pl.ds(col0(f0 + 1), _TC)], tbufB, semB)
        pltpu.make_async_copy(wt_hbm.at[:, pl.ds(0, _TC)], tbufA, semA).wait()
        transpose_chunk(tbufA, _TC, col0(f0))

        @pl.when(f0 + 2 < _CPT)
        def _():
            pltpu.async_copy(wt_hbm.at[:, pl.ds(col0(f0 + 2), _TC)], tbufA, semA)

        pltpu.make_async_copy(wt_hbm.at[:, pl.ds(0, _TC)], tbufB, semB).wait()
        transpose_chunk(tbufB, _TC, col0(f0 + 1))
        return carry

    lax.fori_loop(0, (_CPT - 1) // 2, pair_body, 0)
    pltpu.make_async_copy(wt_hbm.at[:, pl.ds(0, _TC)], tbufA, semA).wait()
    transpose_chunk(tbufA, _TC, col0(_CPT - 1))

    @pl.when(s == 0)
    def _():
        pltpu.sync_copy(tail_hbm, tbufA)
        transpose_chunk(tbufA, _TAIL, _NCHUNK * _TC)


def _lookup_body(idT_hbm, valT_hbm, wrm_hbm, out_hbm,
                 idbuf, valbuf, rowsA, rowsB, obuf, semA, semB):
    c = lax.axis_index("c")
    s = lax.axis_index("s")
    wid = c * _NS + s
    b0 = wid * _BW
    lanes = lax.iota(jnp.int32, 16)

    pltpu.sync_copy(idT_hbm.at[:, pl.ds(b0, _BW)], idbuf)
    pltpu.sync_copy(valT_hbm.at[:, pl.ds(b0, _BW)], valbuf)

    def gather_f(f, rows, sem):
        pltpu.async_copy(wrm_hbm.at[c].at[idbuf.at[f]], rows, sem)

    def compute_f(f, rows):
        def g_body(g, carry):
            bidx = g * 16 + lanes
            v16 = valbuf[f, pl.ds(g * 16, 16)]
            for e in range(_E):
                col = jnp.full((16,), e, jnp.int32)
                obuf[e, pl.ds(g * 16, 16)] = plsc.load_gather(rows, [bidx, col]) * v16
            return carry
        lax.fori_loop(0, _BW // 16, g_body, 0)
        pltpu.sync_copy(obuf, out_hbm.at[f, :, pl.ds(b0, _BW)])

    gather_f(0, rowsA, semA)

    def f_pair(i, carry):
        fA = 2 * i
        gather_f(fA + 1, rowsB, semB)
        pltpu.make_async_copy(wrm_hbm.at[c].at[idbuf.at[0]], rowsA, semA).wait()
        compute_f(fA, rowsA)

        @pl.when(fA + 2 < _F)
        def _():
            gather_f(fA + 2, rowsA, semA)

        pltpu.make_async_copy(wrm_hbm.at[c].at[idbuf.at[0]], rowsB, semB).wait()
        compute_f(fA + 1, rowsB)
        return carry

    lax.fori_loop(0, _F // 2, f_pair, 0)


def kernel(id, value, W):
    idT = id.T
    valT = value.T
    wT = W.T
    tail = jnp.pad(lax.slice(wT, (0, _NCHUNK * _TC), (_E, _V)),
                   ((0, 0), (0, _TC - _TAIL)))
    mesh = plsc.VectorSubcoreMesh(core_axis_name="c", subcore_axis_name="s")

    wrm128 = pl.kernel(
        _transpose_body,
        mesh=mesh,
        compiler_params=pltpu.CompilerParams(
            use_tc_tiling_on_sc=True, needs_layout_passes=False),
        out_type=jax.ShapeDtypeStruct((_NC, _VP * _E // 128, 128), jnp.float32),
        scratch_types=[
            pltpu.VMEM((_E, _TC), jnp.float32),
            pltpu.VMEM((_E, _TC), jnp.float32),
            pltpu.VMEM((_TC * _E // 128, 128), jnp.float32),
            pltpu.SemaphoreType.DMA,
            pltpu.SemaphoreType.DMA,
        ],
    )(wT, tail)
    wrm = wrm128.reshape(_NC, _VP, _E)

    outT = pl.kernel(
        _lookup_body,
        mesh=mesh,
        compiler_params=pltpu.CompilerParams(
            use_tc_tiling_on_sc=False, needs_layout_passes=False),
        out_type=jax.ShapeDtypeStruct((_F, _E, _B), jnp.float32),
        scratch_types=[
            pltpu.VMEM((_F, _BW), jnp.int32),
            pltpu.VMEM((_F, _BW), jnp.float32),
            pltpu.VMEM((_BW, _E), jnp.float32),
            pltpu.VMEM((_BW, _E), jnp.float32),
            pltpu.VMEM((_E, _BW), jnp.float32),
            pltpu.SemaphoreType.DMA,
            pltpu.SemaphoreType.DMA,
        ],
    )(idT, valT, wrm)
    return outT.transpose(2, 0, 1)


# speedup vs baseline: 5.4459x; 2.3849x over previous
import jax
import jax.numpy as jnp
from jax import lax
from jax.experimental import pallas as pl
from jax.experimental.pallas import tpu as pltpu
from jax.experimental.pallas import tpu_sc as plsc

_B = 16384
_F = 26
_E = 16
_V = 1000000
_VP = 1000064

_NC = 2
_NS = 16
_NW = _NC * _NS
_BW = _B // _NW

_TC = 1024
_NCHUNK = _V // _TC
_TAIL = _V - _NCHUNK * _TC
_CPW = _NCHUNK // _NW


def _transpose_body(wt_hbm, tail_hbm, wrm_hbm,
                    tbufA, tbufB, tobufA, tobufB, semA, semB, osemA, osemB):
    c = lax.axis_index("c")
    s = lax.axis_index("s")
    w = c * _NS + s
    lanes = lax.iota(jnp.int32, 16)

    nchunks = jnp.where(w < 16, 31, 30)

    def col0(j):
        return (j * _NW + w) * _TC

    def transpose_chunk(tbuf, tobuf, n):
        def i_body(i, carry):
            vecs = []
            for jj in range(8):
                q = i * 8 + jj
                vecs.append(plsc.load_gather(
                    tbuf, [lanes, jnp.full((16,), q, jnp.int32)]))
            for jj in range(8):
                tobuf[i, pl.ds(jj * 16, 16)] = vecs[jj]
            return carry
        lax.fori_loop(0, n * _E // 128, i_body, 0)

    def out_copy(tobuf, osem, n, cbase):
        r0 = pl.multiple_of(cbase * _E // 128, 8)
        pltpu.async_copy(tobuf.at[pl.ds(0, n * _E // 128), :],
                         wrm_hbm.at[pl.ds(r0, n * _E // 128), :], osem)

    def out_wait(tobuf, osem, n):
        pltpu.make_async_copy(tobuf.at[pl.ds(0, n * _E // 128), :],
                              wrm_hbm.at[pl.ds(0, n * _E // 128), :], osem).wait()

    pltpu.async_copy(wt_hbm.at[:, pl.ds(col0(0), _TC)], tbufA, semA)

    def pair_body(j, carry):
        f0 = 2 * j

        @pl.when(f0 < nchunks)
        def _():
            @pl.when(f0 + 1 < nchunks)
            def _():
                pltpu.async_copy(wt_hbm.at[:, pl.ds(col0(f0 + 1), _TC)],
                                 tbufB, semB)
            pltpu.make_async_copy(wt_hbm.at[:, pl.ds(0, _TC)], tbufA, semA).wait()

            @pl.when(j > 0)
            def _():
                out_wait(tobufA, osemA, _TC)
            transpose_chunk(tbufA, tobufA, _TC)
            out_copy(tobufA, osemA, _TC, col0(f0))

            @pl.when(f0 + 2 < nchunks)
            def _():
                pltpu.async_copy(wt_hbm.at[:, pl.ds(col0(f0 + 2), _TC)],
                                 tbufA, semA)

        @pl.when(f0 + 1 < nchunks)
        def _():
            pltpu.make_async_copy(wt_hbm.at[:, pl.ds(0, _TC)], tbufB, semB).wait()

            @pl.when(j > 0)
            def _():
                out_wait(tobufB, osemB, _TC)
            transpose_chunk(tbufB, tobufB, _TC)
            out_copy(tobufB, osemB, _TC, col0(f0 + 1))
        return carry

    lax.fori_loop(0, 16, pair_body, 0)
    out_wait(tobufA, osemA, _TC)
    out_wait(tobufB, osemB, _TC)

    @pl.when(w == 16)
    def _():
        pltpu.sync_copy(tail_hbm, tbufA)
        transpose_chunk(tbufA, tobufA, _TAIL)
        out_copy(tobufA, osemA, _TAIL, _NCHUNK * _TC)
        out_wait(tobufA, osemA, _TAIL)


def _lookup_body(idT_hbm, valT_hbm, wrm_hbm, out_hbm,
                 idbuf, valbuf, rowsA, rowsB, obuf, semA, semB):
    c = lax.axis_index("c")
    s = lax.axis_index("s")
    wid = c * _NS + s
    b0 = wid * _BW
    lanes = lax.iota(jnp.int32, 16)

    pltpu.sync_copy(idT_hbm.at[:, pl.ds(b0, _BW)], idbuf)
    pltpu.sync_copy(valT_hbm.at[:, pl.ds(b0, _BW)], valbuf)

    def gather_f(f, rows, sem):
        pltpu.async_copy(wrm_hbm.at[idbuf.at[f]], rows, sem)

    def compute_f(f, rows):
        def g_body(g, carry):
            bidx = g * 16 + lanes
            v16 = valbuf[f, pl.ds(g * 16, 16)]
            for e in range(_E):
                col = jnp.full((16,), e, jnp.int32)
                obuf[e, pl.ds(g * 16, 16)] = plsc.load_gather(rows, [bidx, col]) * v16
            return carry
        lax.fori_loop(0, _BW // 16, g_body, 0)
        pltpu.sync_copy(obuf, out_hbm.at[f, :, pl.ds(b0, _BW)])

    gather_f(0, rowsA, semA)

    def f_pair(i, carry):
        fA = 2 * i
        gather_f(fA + 1, rowsB, semB)
        pltpu.make_async_copy(wrm_hbm.at[idbuf.at[0]], rowsA, semA).wait()
        compute_f(fA, rowsA)

        @pl.when(fA + 2 < _F)
        def _():
            gather_f(fA + 2, rowsA, semA)

        pltpu.make_async_copy(wrm_hbm.at[idbuf.at[0]], rowsB, semB).wait()
        compute_f(fA + 1, rowsB)
        return carry

    lax.fori_loop(0, _F // 2, f_pair, 0)


def kernel(id, value, W):
    idT = id.T
    valT = value.T
    wT = W.T
    tail = jnp.pad(lax.slice(wT, (0, _NCHUNK * _TC), (_E, _V)),
                   ((0, 0), (0, _TC - _TAIL)))
    mesh = plsc.VectorSubcoreMesh(core_axis_name="c", subcore_axis_name="s")

    wrm128 = pl.kernel(
        _transpose_body,
        mesh=mesh,
        compiler_params=pltpu.CompilerParams(
            use_tc_tiling_on_sc=True, needs_layout_passes=False),
        out_type=jax.ShapeDtypeStruct((_VP * _E // 128, 128), jnp.float32),
        scratch_types=[
            pltpu.VMEM((_E, _TC), jnp.float32),
            pltpu.VMEM((_E, _TC), jnp.float32),
            pltpu.VMEM((_TC * _E // 128, 128), jnp.float32),
            pltpu.VMEM((_TC * _E // 128, 128), jnp.float32),
            pltpu.SemaphoreType.DMA,
            pltpu.SemaphoreType.DMA,
            pltpu.SemaphoreType.DMA,
            pltpu.SemaphoreType.DMA,
        ],
    )(wT, tail)
    wrm = wrm128.reshape(_VP, _E)

    outT = pl.kernel(
        _lookup_body,
        mesh=mesh,
        compiler_params=pltpu.CompilerParams(
            use_tc_tiling_on_sc=False, needs_layout_passes=False),
        out_type=jax.ShapeDtypeStruct((_F, _E, _B), jnp.float32),
        scratch_types=[
            pltpu.VMEM((_F, _BW), jnp.int32),
            pltpu.VMEM((_F, _BW), jnp.float32),
            pltpu.VMEM((_BW, _E), jnp.float32),
            pltpu.VMEM((_BW, _E), jnp.float32),
            pltpu.VMEM((_E, _BW), jnp.float32),
            pltpu.SemaphoreType.DMA,
            pltpu.SemaphoreType.DMA,
        ],
    )(idT, valT, wrm)
    return outT.transpose(2, 0, 1)


# speedup vs baseline: 7.5791x; 1.3917x over previous
import jax
import jax.numpy as jnp
from jax import lax
from jax.experimental import pallas as pl
from jax.experimental.pallas import tpu as pltpu
from jax.experimental.pallas import tpu_sc as plsc

_B = 16384
_F = 26
_E = 16
_V = 1000000
_VP = 1000064

_NC = 2
_NS = 16
_NW = _NC * _NS
_BW = _B // _NW

_TC = 1024
_NCHUNK = _V // _TC
_TAIL = _V - _NCHUNK * _TC
_CPW = _NCHUNK // _NW


def _transpose_body(wt_hbm, tail_hbm, wrm_hbm,
                    tbufA, tbufB, tobufA, tobufB, semA, semB, osemA, osemB):
    c = lax.axis_index("c")
    s = lax.axis_index("s")
    w = c * _NS + s
    lanes = lax.iota(jnp.int32, 16)

    nchunks = jnp.where(w < 16, 31, 30)

    def col0(j):
        return (j * _NW + w) * _TC

    def transpose_chunk(tbuf, tobuf, n):
        def q_body(qg, carry):
            q0 = qg * 16
            qv = q0 + lanes
            row_vec = lax.shift_right_logical(qv, 3)
            colbase = (qv & 7) * 16
            for e in range(_E):
                vec = tbuf[e, pl.ds(q0, 16)]
                plsc.store_scatter(tobuf, [row_vec, colbase + e], vec)
            return carry
        lax.fori_loop(0, n // 16, q_body, 0)

    def out_copy(tobuf, osem, n, cbase):
        r0 = pl.multiple_of(cbase * _E // 128, 8)
        pltpu.async_copy(tobuf.at[pl.ds(0, n * _E // 128), :],
                         wrm_hbm.at[pl.ds(r0, n * _E // 128), :], osem)

    def out_wait(tobuf, osem, n):
        pltpu.make_async_copy(tobuf.at[pl.ds(0, n * _E // 128), :],
                              wrm_hbm.at[pl.ds(0, n * _E // 128), :], osem).wait()

    pltpu.async_copy(wt_hbm.at[:, pl.ds(col0(0), _TC)], tbufA, semA)

    def pair_body(j, carry):
        f0 = 2 * j

        @pl.when(f0 < nchunks)
        def _():
            @pl.when(f0 + 1 < nchunks)
            def _():
                pltpu.async_copy(wt_hbm.at[:, pl.ds(col0(f0 + 1), _TC)],
                                 tbufB, semB)
            pltpu.make_async_copy(wt_hbm.at[:, pl.ds(0, _TC)], tbufA, semA).wait()

            @pl.when(j > 0)
            def _():
                out_wait(tobufA, osemA, _TC)
            transpose_chunk(tbufA, tobufA, _TC)
            out_copy(tobufA, osemA, _TC, col0(f0))

            @pl.when(f0 + 2 < nchunks)
            def _():
                pltpu.async_copy(wt_hbm.at[:, pl.ds(col0(f0 + 2), _TC)],
                                 tbufA, semA)

        @pl.when(f0 + 1 < nchunks)
        def _():
            pltpu.make_async_copy(wt_hbm.at[:, pl.ds(0, _TC)], tbufB, semB).wait()

            @pl.when(j > 0)
            def _():
                out_wait(tobufB, osemB, _TC)
            transpose_chunk(tbufB, tobufB, _TC)
            out_copy(tobufB, osemB, _TC, col0(f0 + 1))
        return carry

    lax.fori_loop(0, 16, pair_body, 0)
    out_wait(tobufA, osemA, _TC)
    out_wait(tobufB, osemB, _TC)

    @pl.when(w == 16)
    def _():
        pltpu.sync_copy(tail_hbm, tbufA)
        transpose_chunk(tbufA, tobufA, _TAIL)
        out_copy(tobufA, osemA, _TAIL, _NCHUNK * _TC)
        out_wait(tobufA, osemA, _TAIL)


def _lookup_body(idT_hbm, valT_hbm, wrm_hbm, out_hbm,
                 idbuf, valbuf, rowsA, rowsB, obuf, semA, semB):
    c = lax.axis_index("c")
    s = lax.axis_index("s")
    wid = c * _NS + s
    b0 = wid * _BW
    lanes = lax.iota(jnp.int32, 16)

    pltpu.sync_copy(idT_hbm.at[:, pl.ds(b0, _BW)], idbuf)
    pltpu.sync_copy(valT_hbm.at[:, pl.ds(b0, _BW)], valbuf)

    def gather_f(f, rows, sem):
        pltpu.async_copy(wrm_hbm.at[idbuf.at[f]], rows, sem)

    def compute_f(f, rows):
        def g_body(g, carry):
            bidx = g * 16 + lanes
            v16 = valbuf[f, pl.ds(g * 16, 16)]
            for e in range(_E):
                col = jnp.full((16,), e, jnp.int32)
                obuf[e, pl.ds(g * 16, 16)] = plsc.load_gather(rows, [bidx, col]) * v16
            return carry
        lax.fori_loop(0, _BW // 16, g_body, 0)
        pltpu.sync_copy(obuf, out_hbm.at[f, :, pl.ds(b0, _BW)])

    gather_f(0, rowsA, semA)

    def f_pair(i, carry):
        fA = 2 * i
        gather_f(fA + 1, rowsB, semB)
        pltpu.make_async_copy(wrm_hbm.at[idbuf.at[0]], rowsA, semA).wait()
        compute_f(fA, rowsA)

        @pl.when(fA + 2 < _F)
        def _():
            gather_f(fA + 2, rowsA, semA)

        pltpu.make_async_copy(wrm_hbm.at[idbuf.at[0]], rowsB, semB).wait()
        compute_f(fA + 1, rowsB)
        return carry

    lax.fori_loop(0, _F // 2, f_pair, 0)


def kernel(id, value, W):
    idT = id.T
    valT = value.T
    wT = W.T
    tail = jnp.pad(lax.slice(wT, (0, _NCHUNK * _TC), (_E, _V)),
                   ((0, 0), (0, _TC - _TAIL)))
    mesh = plsc.VectorSubcoreMesh(core_axis_name="c", subcore_axis_name="s")

    wrm128 = pl.kernel(
        _transpose_body,
        mesh=mesh,
        compiler_params=pltpu.CompilerParams(
            use_tc_tiling_on_sc=True, needs_layout_passes=False),
        out_type=jax.ShapeDtypeStruct((_VP * _E // 128, 128), jnp.float32),
        scratch_types=[
            pltpu.VMEM((_E, _TC), jnp.float32),
            pltpu.VMEM((_E, _TC), jnp.float32),
            pltpu.VMEM((_TC * _E // 128, 128), jnp.float32),
            pltpu.VMEM((_TC * _E // 128, 128), jnp.float32),
            pltpu.SemaphoreType.DMA,
            pltpu.SemaphoreType.DMA,
            pltpu.SemaphoreType.DMA,
            pltpu.SemaphoreType.DMA,
        ],
    )(wT, tail)
    wrm = wrm128.reshape(_VP, _E)

    outT = pl.kernel(
        _lookup_body,
        mesh=mesh,
        compiler_params=pltpu.CompilerParams(
            use_tc_tiling_on_sc=False, needs_layout_passes=False),
        out_type=jax.ShapeDtypeStruct((_F, _E, _B), jnp.float32),
        scratch_types=[
            pltpu.VMEM((_F, _BW), jnp.int32),
            pltpu.VMEM((_F, _BW), jnp.float32),
            pltpu.VMEM((_BW, _E), jnp.float32),
            pltpu.VMEM((_BW, _E), jnp.float32),
            pltpu.VMEM((_E, _BW), jnp.float32),
            pltpu.SemaphoreType.DMA,
            pltpu.SemaphoreType.DMA,
        ],
    )(idT, valT, wrm)
    return outT.transpose(2, 0, 1)


# speedup vs baseline: 7.7173x; 1.0182x over previous
import jax
import jax.numpy as jnp
from jax import lax
from jax.experimental import pallas as pl
from jax.experimental.pallas import tpu as pltpu
from jax.experimental.pallas import tpu_sc as plsc

_B = 16384
_F = 26
_E = 16
_V = 1000000
_VP = 1000064

_NC = 2
_NS = 16
_NW = _NC * _NS
_BW = _B // _NW

_TC = 1024
_NCHUNK = _V // _TC
_TAIL = _V - _NCHUNK * _TC
_CPW = _NCHUNK // _NW


def _transpose_body(wt_hbm, tail_hbm, wrm_hbm,
                    tbufA, tbufB, tobufA, tobufB, semA, semB, osemA, osemB):
    c = lax.axis_index("c")
    s = lax.axis_index("s")
    w = c * _NS + s
    lanes = lax.iota(jnp.int32, 16)

    nchunks = jnp.where(w < 16, 31, 30)

    def col0(j):
        return (j * _NW + w) * _TC

    def transpose_chunk(tbuf, tobuf, n):
        def q_body(qg, carry):
            q0 = qg * 16
            qv = q0 + lanes
            row_vec = lax.shift_right_logical(qv, 3)
            colbase = (qv & 7) * 16
            for e in range(_E):
                vec = tbuf[e, pl.ds(q0, 16)]
                plsc.store_scatter(tobuf, [row_vec, colbase + e], vec)
            return carry
        lax.fori_loop(0, n // 16, q_body, 0)

    def out_copy(tobuf, osem, n, cbase):
        r0 = pl.multiple_of(cbase * _E // 128, 8)
        pltpu.async_copy(tobuf.at[pl.ds(0, n * _E // 128), :],
                         wrm_hbm.at[pl.ds(r0, n * _E // 128), :], osem)

    def out_wait(tobuf, osem, n):
        pltpu.make_async_copy(tobuf.at[pl.ds(0, n * _E // 128), :],
                              wrm_hbm.at[pl.ds(0, n * _E // 128), :], osem).wait()

    pltpu.async_copy(wt_hbm.at[:, pl.ds(col0(0), _TC)], tbufA, semA)

    def pair_body(j, carry):
        f0 = 2 * j

        @pl.when(f0 < nchunks)
        def _():
            @pl.when(f0 + 1 < nchunks)
            def _():
                pltpu.async_copy(wt_hbm.at[:, pl.ds(col0(f0 + 1), _TC)],
                                 tbufB, semB)
            pltpu.make_async_copy(wt_hbm.at[:, pl.ds(0, _TC)], tbufA, semA).wait()

            @pl.when(j > 0)
            def _():
                out_wait(tobufA, osemA, _TC)
            transpose_chunk(tbufA, tobufA, _TC)
            out_copy(tobufA, osemA, _TC, col0(f0))

            @pl.when(f0 + 2 < nchunks)
            def _():
                pltpu.async_copy(wt_hbm.at[:, pl.ds(col0(f0 + 2), _TC)],
                                 tbufA, semA)

        @pl.when(f0 + 1 < nchunks)
        def _():
            pltpu.make_async_copy(wt_hbm.at[:, pl.ds(0, _TC)], tbufB, semB).wait()

            @pl.when(j > 0)
            def _():
                out_wait(tobufB, osemB, _TC)
            transpose_chunk(tbufB, tobufB, _TC)
            out_copy(tobufB, osemB, _TC, col0(f0 + 1))
        return carry

    lax.fori_loop(0, 16, pair_body, 0)
    out_wait(tobufA, osemA, _TC)
    out_wait(tobufB, osemB, _TC)

    @pl.when(w == 16)
    def _():
        pltpu.sync_copy(tail_hbm, tbufA)
        transpose_chunk(tbufA, tobufA, _TAIL)
        out_copy(tobufA, osemA, _TAIL, _NCHUNK * _TC)
        out_wait(tobufA, osemA, _TAIL)


def _lookup_body(idT_hbm, valT_hbm, wrm_hbm, out_hbm,
                 idbuf, valbuf, rows4, obufA, obufB,
                 gsem0, gsem1, gsem2, gsem3, osemA, osemB):
    c = lax.axis_index("c")
    s = lax.axis_index("s")
    wid = c * _NS + s
    b0 = wid * _BW
    lanes = lax.iota(jnp.int32, 16)
    gsems = (gsem0, gsem1, gsem2, gsem3)
    obufs = (obufA, obufB)
    osems = (osemA, osemB)

    pltpu.sync_copy(idT_hbm.at[:, pl.ds(b0, _BW)], idbuf)
    pltpu.sync_copy(valT_hbm.at[:, pl.ds(b0, _BW)], valbuf)

    def gather_f(f, k):
        pltpu.async_copy(wrm_hbm.at[idbuf.at[f]], rows4.at[k], gsems[k])

    def gwait(k):
        pltpu.make_async_copy(wrm_hbm.at[idbuf.at[0]], rows4.at[k], gsems[k]).wait()

    def owait(m):
        pltpu.make_async_copy(obufs[m], out_hbm.at[0, :, pl.ds(b0, _BW)],
                              osems[m]).wait()

    def compute_f(f, k, m):
        obuf = obufs[m]

        def g_body(g, carry):
            bidx = g * 16 + lanes
            v16 = valbuf[f, pl.ds(g * 16, 16)]
            for e in range(_E):
                col = jnp.full((16,), e, jnp.int32)
                obuf[e, pl.ds(g * 16, 16)] = (
                    plsc.load_gather(rows4.at[k], [bidx, col]) * v16)
            return carry
        lax.fori_loop(0, _BW // 16, g_body, 0)
        pltpu.async_copy(obuf, out_hbm.at[f, :, pl.ds(b0, _BW)], osems[m])

    for k in range(4):
        gather_f(k, k)

    def round4(p, carry):
        for k in range(4):
            r = 4 * p + k
            m = k % 2
            gwait(k)

            @pl.when(r >= 2)
            def _():
                owait(m)
            compute_f(r, k, m)

            @pl.when(r + 4 < _F)
            def _():
                gather_f(r + 4, k)
        return carry

    lax.fori_loop(0, (_F - 2) // 4, round4, 0)
    for k in range(2):
        r = _F - 2 + k
        gwait(k)
        owait(k)
        compute_f(r, k, k)
    owait(0)
    owait(1)


def kernel(id, value, W):
    idT = id.T
    valT = value.T
    wT = W.T
    tail = jnp.pad(lax.slice(wT, (0, _NCHUNK * _TC), (_E, _V)),
                   ((0, 0), (0, _TC - _TAIL)))
    mesh = plsc.VectorSubcoreMesh(core_axis_name="c", subcore_axis_name="s")

    wrm128 = pl.kernel(
        _transpose_body,
        mesh=mesh,
        compiler_params=pltpu.CompilerParams(
            use_tc_tiling_on_sc=True, needs_layout_passes=False),
        out_type=jax.ShapeDtypeStruct((_VP * _E // 128, 128), jnp.float32),
        scratch_types=[
            pltpu.VMEM((_E, _TC), jnp.float32),
            pltpu.VMEM((_E, _TC), jnp.float32),
            pltpu.VMEM((_TC * _E // 128, 128), jnp.float32),
            pltpu.VMEM((_TC * _E // 128, 128), jnp.float32),
            pltpu.SemaphoreType.DMA,
            pltpu.SemaphoreType.DMA,
            pltpu.SemaphoreType.DMA,
            pltpu.SemaphoreType.DMA,
        ],
    )(wT, tail)
    wrm = wrm128.reshape(_VP, _E)

    outT = pl.kernel(
        _lookup_body,
        mesh=mesh,
        compiler_params=pltpu.CompilerParams(
            use_tc_tiling_on_sc=False, needs_layout_passes=False),
        out_type=jax.ShapeDtypeStruct((_F, _E, _B), jnp.float32),
        scratch_types=[
            pltpu.VMEM((_F, _BW), jnp.int32),
            pltpu.VMEM((_F, _BW), jnp.float32),
            pltpu.VMEM((4, _BW, _E), jnp.float32),
            pltpu.VMEM((_E, _BW), jnp.float32),
            pltpu.VMEM((_E, _BW), jnp.float32),
            pltpu.SemaphoreType.DMA,
            pltpu.SemaphoreType.DMA,
            pltpu.SemaphoreType.DMA,
            pltpu.SemaphoreType.DMA,
            pltpu.SemaphoreType.DMA,
            pltpu.SemaphoreType.DMA,
        ],
    )(idT, valT, wrm)
    return outT.transpose(2, 0, 1)
